# baseline (speedup 1.0000x reference)
import jax
import jax.numpy as jnp
from jax import lax
from jax.experimental import pallas as pl
from jax.experimental.pallas import tpu as pltpu

_NUM_HEAD = 4
_NUM_EMB = 1024
_EMB_DIM = 256
_COMMIT = 0.25
_BN = 512


def _vq_block(x_ref, emb_ref, q_ref, enc_ref, loss_ref, perp_ref,
              e2_scr, cnt_scr, loss_scr):
    i = pl.program_id(0)
    nblk = pl.num_programs(0)
    bn = x_ref.shape[0]

    @pl.when(i == 0)
    def _init():
        e2_scr[...] = jnp.sum(emb_ref[...] ** 2, axis=2)
        cnt_scr[...] = jnp.zeros_like(cnt_scr)
        loss_scr[...] = jnp.zeros_like(loss_scr)

    x = x_ref[...]
    loss_acc = loss_scr[...]
    cnt_acc = cnt_scr[...]
    for h in range(_NUM_HEAD):
        xh = x[:, h * _EMB_DIM:(h + 1) * _EMB_DIM]
        m = lax.dot_general(xh + xh, emb_ref[h],
                            (((1,), (1,)), ((), ())))
        x2 = jnp.sum(xh * xh, axis=1, keepdims=True)
        d = (x2 + e2_scr[h][None, :]) - m
        dmin = jnp.min(d, axis=1, keepdims=True)
        onehot = jnp.where(d <= dmin, 1.0, 0.0).astype(jnp.float32)
        enc_ref[:, h, :] = onehot
        qh = jnp.dot(onehot, emb_ref[h])
        q_ref[:, h * _EMB_DIM:(h + 1) * _EMB_DIM] = qh
        sq = (qh - xh) ** 2
        loss_acc = loss_acc + jnp.sum(sq.reshape(bn // 8, 8, _EMB_DIM), axis=0)
        cnt_acc = cnt_acc + jnp.sum(onehot.reshape(bn // 8, 8, _NUM_EMB), axis=0)
    loss_scr[...] = loss_acc
    cnt_scr[...] = cnt_acc

    @pl.when(i == nblk - 1)
    def _fin():
        n_tok = bn * nblk
        loss_val = _COMMIT * jnp.sum(loss_scr[...]) / jnp.float32(n_tok * _NUM_HEAD * _EMB_DIM)
        loss_ref[...] = jnp.reshape(loss_val, (1, 1))
        avg = jnp.sum(cnt_scr[...], axis=0, keepdims=True) / jnp.float32(n_tok * _NUM_HEAD)
        perp_val = jnp.exp(-jnp.sum(avg * jnp.log(avg + 1e-10)))
        perp_ref[...] = jnp.reshape(perp_val, (1, 1))


def kernel(inputs, embedding):
    B, T, F = inputs.shape
    n_tok = B * T
    flat = inputs.reshape(n_tok, F)
    grid = n_tok // _BN

    q, enc, loss, perp = pl.pallas_call(
        _vq_block,
        grid=(grid,),
        in_specs=[
            pl.BlockSpec((_BN, F), lambda i: (i, 0)),
            pl.BlockSpec((_NUM_HEAD, _NUM_EMB, _EMB_DIM), lambda i: (0, 0, 0)),
        ],
        out_specs=[
            pl.BlockSpec((_BN, F), lambda i: (i, 0)),
            pl.BlockSpec((_BN, _NUM_HEAD, _NUM_EMB), lambda i: (i, 0, 0)),
            pl.BlockSpec((1, 1), lambda i: (0, 0)),
            pl.BlockSpec((1, 1), lambda i: (0, 0)),
        ],
        out_shape=[
            jax.ShapeDtypeStruct((n_tok, F), jnp.float32),
            jax.ShapeDtypeStruct((n_tok, _NUM_HEAD, _NUM_EMB), jnp.float32),
            jax.ShapeDtypeStruct((1, 1), jnp.float32),
            jax.ShapeDtypeStruct((1, 1), jnp.float32),
        ],
        scratch_shapes=[
            pltpu.VMEM((_NUM_HEAD, _NUM_EMB), jnp.float32),
            pltpu.VMEM((8, _NUM_EMB), jnp.float32),
            pltpu.VMEM((8, _EMB_DIM), jnp.float32),
        ],
        compiler_params=pltpu.CompilerParams(
            dimension_semantics=("arbitrary",),
        ),
    )(flat, embedding)

    loss_out = loss.reshape(())
    perp_out = perp.reshape(())
    q_out = q.reshape(B, T, _NUM_HEAD, _EMB_DIM)
    enc_out = enc.reshape(B, T, _NUM_HEAD, _NUM_EMB)
    return (loss_out, q_out, perp_out, enc_out)


# speedup vs baseline: 1.3416x; 1.3416x over previous
import jax
import jax.numpy as jnp
from jax import lax
from jax.experimental import pallas as pl
from jax.experimental.pallas import tpu as pltpu

_NUM_HEAD = 4
_NUM_EMB = 1024
_EMB_DIM = 256
_COMMIT = 0.25
_BN = 512


def _vq_block(x_ref, emb_ref, q_ref, enc_ref, loss_ref, perp_ref,
              e2_scr, cnt_scr, loss_scr):
    i = pl.program_id(0)
    nblk = pl.num_programs(0)
    bn = x_ref.shape[0]

    @pl.when(i == 0)
    def _init():
        e2_scr[...] = jnp.sum(emb_ref[...] ** 2, axis=2)
        cnt_scr[...] = jnp.zeros_like(cnt_scr)
        loss_scr[...] = jnp.zeros_like(loss_scr)

    x = x_ref[...]
    loss_acc = loss_scr[...]
    cnt_acc = cnt_scr[...]
    for h in range(_NUM_HEAD):
        xh = x[:, h * _EMB_DIM:(h + 1) * _EMB_DIM]
        m = lax.dot_general(xh + xh, emb_ref[h],
                            (((1,), (1,)), ((), ())))
        x2 = jnp.sum(xh * xh, axis=1, keepdims=True)
        d = (x2 + e2_scr[h][None, :]) - m
        dmin = jnp.min(d, axis=1, keepdims=True)
        onehot = jnp.where(d <= dmin, 1.0, 0.0).astype(jnp.float32)
        enc_ref[:, h, :] = onehot
        qh = jnp.dot(onehot, emb_ref[h])
        q_ref[:, h, :] = qh
        loss_acc = loss_acc + jnp.sum(dmin.reshape(bn // 8, 8, 1), axis=0)
        cnt_acc = cnt_acc + jnp.sum(onehot.reshape(bn // 8, 8, _NUM_EMB), axis=0)
    loss_scr[...] = loss_acc
    cnt_scr[...] = cnt_acc

    @pl.when(i == nblk - 1)
    def _fin():
        n_tok = bn * nblk
        loss_val = _COMMIT * jnp.sum(loss_scr[...]) / jnp.float32(n_tok * _NUM_HEAD * _EMB_DIM)
        loss_ref[...] = jnp.reshape(loss_val, (1, 1))
        avg = jnp.sum(cnt_scr[...], axis=0, keepdims=True) / jnp.float32(n_tok * _NUM_HEAD)
        perp_val = jnp.exp(-jnp.sum(avg * jnp.log(avg + 1e-10)))
        perp_ref[...] = jnp.reshape(perp_val, (1, 1))


def kernel(inputs, embedding):
    B, T, F = inputs.shape
    n_tok = B * T
    flat = inputs.reshape(n_tok, F)
    grid = n_tok // _BN

    q, enc, loss, perp = pl.pallas_call(
        _vq_block,
        grid=(grid,),
        in_specs=[
            pl.BlockSpec((_BN, F), lambda i: (i, 0)),
            pl.BlockSpec((_NUM_HEAD, _NUM_EMB, _EMB_DIM), lambda i: (0, 0, 0)),
        ],
        out_specs=[
            pl.BlockSpec((_BN, _NUM_HEAD, _EMB_DIM), lambda i: (i, 0, 0)),
            pl.BlockSpec((_BN, _NUM_HEAD, _NUM_EMB), lambda i: (i, 0, 0)),
            pl.BlockSpec((1, 1), lambda i: (0, 0)),
            pl.BlockSpec((1, 1), lambda i: (0, 0)),
        ],
        out_shape=[
            jax.ShapeDtypeStruct((n_tok, _NUM_HEAD, _EMB_DIM), jnp.float32),
            jax.ShapeDtypeStruct((n_tok, _NUM_HEAD, _NUM_EMB), jnp.float32),
            jax.ShapeDtypeStruct((1, 1), jnp.float32),
            jax.ShapeDtypeStruct((1, 1), jnp.float32),
        ],
        scratch_shapes=[
            pltpu.VMEM((_NUM_HEAD, _NUM_EMB), jnp.float32),
            pltpu.VMEM((8, _NUM_EMB), jnp.float32),
            pltpu.VMEM((8, 1), jnp.float32),
        ],
        compiler_params=pltpu.CompilerParams(
            dimension_semantics=("arbitrary",),
        ),
    )(flat, embedding)

    loss_out = loss.reshape(())
    perp_out = perp.reshape(())
    q_out = q.reshape(B, T, _NUM_HEAD, _EMB_DIM)
    enc_out = enc.reshape(B, T, _NUM_HEAD, _NUM_EMB)
    return (loss_out, q_out, perp_out, enc_out)
